# Initial kernel scaffold; baseline (speedup 1.0000x reference)
import jax
import jax.numpy as jnp
from jax.experimental import pallas as pl


def kernel(x, edge_index, edge_attr, l0_lin_W, l0_lin_b, l0_linA_W, l0_linA_b, l0_linB_W, l0_linB_b, l1_bn_gamma, l1_bn_beta, l1_lin_W, l1_lin_b, l1_linA_W, l1_linA_b, l1_linB_W, l1_linB_b, fe_W, fe_b):
    raise NotImplementedError("write your pallas kernel here")


# speedup vs baseline: 4.1719x; 4.1719x over previous
import functools

import jax
import jax.numpy as jnp
from jax import lax
from jax.experimental import pallas as pl
from jax.experimental.pallas import tpu as pltpu
from jax.experimental.pallas import tpu_sc as plsc

N = 10000
E = 320000
D = 128
HK = 16
EMB = 64
EPS = 1e-5

NC = 2
NS = 16
NW = NC * NS
CE = 128
EPT = 10112
NCHUNK = EPT // CE
EPAD = NW * EPT
NPAD = 10240
ROWS_PER_TILE = NPAD // NS

RB = 1000
GRID = N // RB


def _edge_body(src_hbm, dst_hbm, ea_hbm, a_hbm, b_hbm, x_hbm, cl_hbm,
               out_hbm,
               sidx, didx, eav, ad, bs, xbuf, wv, clv, zbuf,
               aggr, sem_a, sem_b, sem_x):
    c = lax.axis_index("c")
    s = lax.axis_index("s")
    wid = c * NS + s
    ebase = wid * EPT

    pltpu.sync_copy(cl_hbm, clv)
    cvec = clv[0, :]
    lvec = clv[1, :]
    lbb_s = clv[2, :][0]

    def _zrow(i, carry):
        for r in range(D // 16):
            zbuf[i, pl.ds(r * 16, 16)] = jnp.zeros((16,), jnp.float32)
        return carry
    lax.fori_loop(0, 128, _zrow, 0)

    def _zcp(j, carry):
        pltpu.sync_copy(zbuf, aggr.at[pl.ds(s * ROWS_PER_TILE + j * 128, 128)])
        return carry
    lax.fori_loop(0, ROWS_PER_TILE // 128, _zcp, 0)
    plsc.subcore_barrier()

    def _chunk(i, carry):
        base = ebase + i * CE
        pltpu.sync_copy(src_hbm.at[pl.ds(base, CE)], sidx)
        pltpu.sync_copy(dst_hbm.at[pl.ds(base, CE)], didx)
        pltpu.sync_copy(ea_hbm.at[pl.ds(base, CE)], eav)
        ca = pltpu.async_copy(a_hbm.at[didx], ad, sem_a)
        cb = pltpu.async_copy(b_hbm.at[sidx], bs, sem_b)
        cx = pltpu.async_copy(x_hbm.at[sidx], xbuf, sem_x)
        ca.wait()
        cb.wait()

        def _grp(g, carry2):
            o = g * 16
            sv = sidx[pl.ds(o, 16)]
            dv = didx[pl.ds(o, 16)]
            ev = eav[pl.ds(o, 16)]
            rows = lax.iota(jnp.int32, 16) + o
            logit = jnp.zeros((16,), jnp.float32) + lbb_s
            for k in range(HK):
                colk = jnp.full((16,), k, jnp.int32)
                hk = (plsc.load_gather(ad, [rows, colk])
                      + plsc.load_gather(bs, [rows, colk])
                      + ev * cvec[k])
                hk = jnp.maximum(hk, 0.0)
                logit = logit + hk * lvec[k]
            w = 1.0 / (1.0 + jnp.exp(-logit))
            w = jnp.where(sv != dv, w, 0.0)
            wv[pl.ds(o, 16)] = w
            return carry2
        lax.fori_loop(0, CE // 16, _grp, 0)

        cx.wait()

        def _scl(g, carry2):
            o = g * 16
            wvec = wv[pl.ds(o, 16)]
            for e in range(16):
                w_s = wvec[e]
                for r in range(D // 16):
                    xbuf[o + e, pl.ds(r * 16, 16)] = (
                        xbuf[o + e, pl.ds(r * 16, 16)] * w_s)
            return carry2
        lax.fori_loop(0, CE // 16, _scl, 0)

        pltpu.sync_copy(xbuf, aggr.at[didx], add=True)
        return carry
    lax.fori_loop(0, NCHUNK, _chunk, 0)

    plsc.subcore_barrier()
    pltpu.sync_copy(aggr.at[pl.ds(s * ROWS_PER_TILE, ROWS_PER_TILE)],
                    out_hbm.at[c, pl.ds(s * ROWS_PER_TILE, ROWS_PER_TILE)])


_edge_aggregate = functools.partial(
    pl.kernel,
    out_type=jax.ShapeDtypeStruct((NC, NPAD, D), jnp.float32),
    mesh=plsc.VectorSubcoreMesh(core_axis_name="c", subcore_axis_name="s"),
    compiler_params=pltpu.CompilerParams(needs_layout_passes=False,
                                         use_tc_tiling_on_sc=False),
    scratch_types=[
        pltpu.VMEM((CE,), jnp.int32),
        pltpu.VMEM((CE,), jnp.int32),
        pltpu.VMEM((CE,), jnp.float32),
        pltpu.VMEM((CE, HK), jnp.float32),
        pltpu.VMEM((CE, HK), jnp.float32),
        pltpu.VMEM((CE, D), jnp.float32),
        pltpu.VMEM((CE,), jnp.float32),
        pltpu.VMEM((3, HK), jnp.float32),
        pltpu.VMEM((128, D), jnp.float32),
        pltpu.VMEM_SHARED((NPAD, D), jnp.float32),
        pltpu.SemaphoreType.DMA,
        pltpu.SemaphoreType.DMA,
        pltpu.SemaphoreType.DMA,
    ],
)(_edge_body)


def _dot(a, b):
    return jnp.dot(a, b, preferred_element_type=jnp.float32)


def _proj_body(x_ref, wd_ref, ws_ref, lab_ref, a_ref, b_ref):
    xb = x_ref[...]
    a_ref[...] = _dot(xb, wd_ref[...]) + lab_ref[...]
    b_ref[...] = _dot(xb, ws_ref[...])


def _proj(x, wdT, wsT, lab):
    return pl.pallas_call(
        _proj_body,
        grid=(GRID,),
        in_specs=[
            pl.BlockSpec((RB, D), lambda i: (i, 0)),
            pl.BlockSpec((D, HK), lambda i: (0, 0)),
            pl.BlockSpec((D, HK), lambda i: (0, 0)),
            pl.BlockSpec((1, HK), lambda i: (0, 0)),
        ],
        out_specs=[
            pl.BlockSpec((RB, HK), lambda i: (i, 0)),
            pl.BlockSpec((RB, HK), lambda i: (i, 0)),
        ],
        out_shape=[
            jax.ShapeDtypeStruct((N, HK), jnp.float32),
            jax.ShapeDtypeStruct((N, HK), jnp.float32),
        ],
    )(x, wdT, wsT, lab)


def _self_gate(xb, wsumT, lab, lb, lbb):
    hs = jnp.maximum(_dot(xb, wsumT) + lab, 0.0)
    logit = jnp.sum(hs * lb, axis=1, keepdims=True) + lbb
    return (1.0 / (1.0 + jnp.exp(-logit))) * xb


def _combine0_body(x_ref, agg_ref, wsum_ref, lab_ref, lb_ref, lbb_ref,
                   ua_ref, ub_ref, b_ref, emb_ref, sums_ref):
    xb = x_ref[...]
    selfm = _self_gate(xb, wsum_ref[...], lab_ref[...], lb_ref[...],
                       lbb_ref[...])
    aggr = agg_ref[0] + agg_ref[1] + selfm
    e = jnp.maximum(_dot(xb, ua_ref[...]) + _dot(aggr, ub_ref[...])
                    + b_ref[...], 0.0)
    emb_ref[...] = e

    @pl.when(pl.program_id(0) == 0)
    def _():
        sums_ref[...] = jnp.zeros_like(sums_ref)
    sums_ref[...] += jnp.concatenate(
        [jnp.sum(e, axis=0, keepdims=True),
         jnp.sum(e * e, axis=0, keepdims=True)], axis=0)


def _combine0(x, agg, wsumT, lab, lb, lbb, ua, ub, b):
    return pl.pallas_call(
        _combine0_body,
        grid=(GRID,),
        in_specs=[
            pl.BlockSpec((RB, D), lambda i: (i, 0)),
            pl.BlockSpec((NC, RB, D), lambda i: (0, i, 0)),
            pl.BlockSpec((D, HK), lambda i: (0, 0)),
            pl.BlockSpec((1, HK), lambda i: (0, 0)),
            pl.BlockSpec((1, HK), lambda i: (0, 0)),
            pl.BlockSpec((1, 1), lambda i: (0, 0)),
            pl.BlockSpec((D, D), lambda i: (0, 0)),
            pl.BlockSpec((D, D), lambda i: (0, 0)),
            pl.BlockSpec((1, D), lambda i: (0, 0)),
        ],
        out_specs=[
            pl.BlockSpec((RB, D), lambda i: (i, 0)),
            pl.BlockSpec((2, D), lambda i: (0, 0)),
        ],
        out_shape=[
            jax.ShapeDtypeStruct((N, D), jnp.float32),
            jax.ShapeDtypeStruct((2, D), jnp.float32),
        ],
    )(x, agg, wsumT, lab, lb, lbb, ua, ub, b)


def _normproj_body(emb_ref, sums_ref, gamma_ref, beta_ref,
                   wd_ref, ws_ref, lab_ref, xn_ref, a_ref, b_ref):
    sums = sums_ref[...]
    mean = sums[0:1] * (1.0 / N)
    var = sums[1:2] * (1.0 / N) - mean * mean
    xb = emb_ref[...]
    xn = ((xb - mean) * lax.rsqrt(var + EPS) * gamma_ref[...]
          + beta_ref[...])
    xn_ref[...] = xn
    a_ref[...] = _dot(xn, wd_ref[...]) + lab_ref[...]
    b_ref[...] = _dot(xn, ws_ref[...])


def _normproj(emb, sums, gamma, beta, wdT, wsT, lab):
    return pl.pallas_call(
        _normproj_body,
        grid=(GRID,),
        in_specs=[
            pl.BlockSpec((RB, D), lambda i: (i, 0)),
            pl.BlockSpec((2, D), lambda i: (0, 0)),
            pl.BlockSpec((1, D), lambda i: (0, 0)),
            pl.BlockSpec((1, D), lambda i: (0, 0)),
            pl.BlockSpec((D, HK), lambda i: (0, 0)),
            pl.BlockSpec((D, HK), lambda i: (0, 0)),
            pl.BlockSpec((1, HK), lambda i: (0, 0)),
        ],
        out_specs=[
            pl.BlockSpec((RB, D), lambda i: (i, 0)),
            pl.BlockSpec((RB, HK), lambda i: (i, 0)),
            pl.BlockSpec((RB, HK), lambda i: (i, 0)),
        ],
        out_shape=[
            jax.ShapeDtypeStruct((N, D), jnp.float32),
            jax.ShapeDtypeStruct((N, HK), jnp.float32),
            jax.ShapeDtypeStruct((N, HK), jnp.float32),
        ],
    )(emb, sums, gamma, beta, wdT, wsT, lab)


def _combine1_body(xn_ref, agg_ref, wsum_ref, lab_ref, lb_ref, lbb_ref,
                   ua_ref, ub_ref, b_ref, few_ref, feb_ref, out_ref):
    xb = xn_ref[...]
    selfm = _self_gate(xb, wsum_ref[...], lab_ref[...], lb_ref[...],
                       lbb_ref[...])
    aggr = agg_ref[0] + agg_ref[1] + selfm
    e = jnp.maximum(_dot(xb, ua_ref[...]) + _dot(aggr, ub_ref[...])
                    + b_ref[...], 0.0)
    out_ref[...] = _dot(e, few_ref[...]) + feb_ref[...]


def _combine1(xn, agg, wsumT, lab, lb, lbb, ua, ub, b, fewT, feb):
    return pl.pallas_call(
        _combine1_body,
        grid=(GRID,),
        in_specs=[
            pl.BlockSpec((RB, D), lambda i: (i, 0)),
            pl.BlockSpec((NC, RB, D), lambda i: (0, i, 0)),
            pl.BlockSpec((D, HK), lambda i: (0, 0)),
            pl.BlockSpec((1, HK), lambda i: (0, 0)),
            pl.BlockSpec((1, HK), lambda i: (0, 0)),
            pl.BlockSpec((1, 1), lambda i: (0, 0)),
            pl.BlockSpec((D, D), lambda i: (0, 0)),
            pl.BlockSpec((D, D), lambda i: (0, 0)),
            pl.BlockSpec((1, D), lambda i: (0, 0)),
            pl.BlockSpec((D, EMB), lambda i: (0, 0)),
            pl.BlockSpec((1, EMB), lambda i: (0, 0)),
        ],
        out_specs=pl.BlockSpec((RB, EMB), lambda i: (i, 0)),
        out_shape=jax.ShapeDtypeStruct((N, EMB), jnp.float32),
    )(xn, agg, wsumT, lab, lb, lbb, ua, ub, b, fewT, feb)


def kernel(x, edge_index, edge_attr,
           l0_lin_W, l0_lin_b, l0_linA_W, l0_linA_b, l0_linB_W, l0_linB_b,
           l1_bn_gamma, l1_bn_beta, l1_lin_W, l1_lin_b, l1_linA_W, l1_linA_b,
           l1_linB_W, l1_linB_b, fe_W, fe_b):
    f32 = jnp.float32
    src = edge_index[0].astype(jnp.int32)
    dst = edge_index[1].astype(jnp.int32)
    ea = edge_attr.astype(f32)
    pad = EPAD - E
    zpad_i = jnp.zeros((pad,), jnp.int32)
    src_p = jnp.concatenate([src, zpad_i])
    dst_p = jnp.concatenate([dst, zpad_i])
    ea_p = jnp.concatenate([ea, jnp.zeros((pad,), f32)])

    def prep(laW, lab, lbW, lbb):
        wdT = laW[:, :D].T
        wsT = laW[:, D:2 * D].T
        wsumT = wdT + wsT
        cvec = laW[:, 2 * D]
        cl = jnp.stack([cvec, lbW[0],
                        jnp.full((HK,), lbb[0], f32)], axis=0)
        lab2 = lab.reshape(1, HK)
        lb2 = lbW.reshape(1, HK)
        lbb2 = lbb.reshape(1, 1)
        return wdT, wsT, wsumT, cl, lab2, lb2, lbb2

    wdT0, wsT0, wsumT0, cl0, lab0, lb0, lbb0 = prep(
        l0_linA_W, l0_linA_b, l0_linB_W, l0_linB_b)
    wdT1, wsT1, wsumT1, cl1, lab1, lb1, lbb1 = prep(
        l1_linA_W, l1_linA_b, l1_linB_W, l1_linB_b)

    ua0 = l0_lin_W[:, :D].T
    ub0 = l0_lin_W[:, D:].T
    b0 = l0_lin_b.reshape(1, D)
    ua1 = l1_lin_W[:, :D].T
    ub1 = l1_lin_W[:, D:].T
    b1 = l1_lin_b.reshape(1, D)
    fewT = fe_W.T
    feb2 = fe_b.reshape(1, EMB)
    gamma = l1_bn_gamma.reshape(1, D)
    beta = l1_bn_beta.reshape(1, D)

    a0, bvec0 = _proj(x, wdT0, wsT0, lab0)
    agg0 = _edge_aggregate(src_p, dst_p, ea_p, a0, bvec0, x, cl0)
    emb0, sums = _combine0(x, agg0, wsumT0, lab0, lb0, lbb0, ua0, ub0, b0)

    xn, a1, bvec1 = _normproj(emb0, sums, gamma, beta, wdT1, wsT1, lab1)
    agg1 = _edge_aggregate(src_p, dst_p, ea_p, a1, bvec1, xn, cl1)
    return _combine1(xn, agg1, wsumT1, lab1, lb1, lbb1, ua1, ub1, b1,
                     fewT, feb2)
